# Initial kernel scaffold; baseline (speedup 1.0000x reference)
import jax
import jax.numpy as jnp
from jax.experimental import pallas as pl


def kernel(positions, obs_positions, sensor_raw, W, b):
    raise NotImplementedError("write your pallas kernel here")


# speedup vs baseline: 1.3881x; 1.3881x over previous
import math

import jax
import jax.numpy as jnp
from jax.experimental import pallas as pl
from jax.experimental.pallas import tpu as pltpu

F = 2048
O = 128
D = 128
NUM_SENSORS = 100
X_MIN = -2.5
X_MAX = 2.5
PNR = 10.0
EXPECTED = 8.0

_BW = (X_MAX - X_MIN) / NUM_SENSORS
_NP = 10.0 ** (-PNR / 10.0)
_INV2BW2 = 1.0 / (2.0 * _BW * _BW)
_INV2NP2 = 1.0 / (2.0 * _NP * _NP)
_REPL = max(1.0, O / EXPECTED)
_CONST = (-math.log(_BW) - 0.5 * math.log(2.0 * math.pi) + math.log(EXPECTED)
          + math.log(X_MAX - X_MIN) - math.log(D - EXPECTED) - math.log(_REPL))

_FB = 32


def _tc_body(pos_ref, obs_ref, sen_ref, w_ref, b_ref, out_ref):
    w = w_ref[0, 0]
    bb = b_ref[0, 0]
    conf = jax.nn.sigmoid((sen_ref[...] - 0.5) * w + bb)
    ct = (2.0 * conf - 1.0) * _INV2NP2 + _CONST
    diff = obs_ref[...][:, :, None] - pos_ref[...][:, None, :]
    out_ref[...] = ct[:, :, None] - (diff * diff) * _INV2BW2


def kernel(positions, obs_positions, sensor_raw, W, b):
    b2 = b.reshape(1, 1)
    grid = (F // _FB,)
    return pl.pallas_call(
        _tc_body,
        grid=grid,
        in_specs=[
            pl.BlockSpec((_FB, O), lambda i: (i, 0)),
            pl.BlockSpec((_FB, D), lambda i: (i, 0)),
            pl.BlockSpec((_FB, D), lambda i: (i, 0)),
            pl.BlockSpec(memory_space=pltpu.SMEM),
            pl.BlockSpec(memory_space=pltpu.SMEM),
        ],
        out_specs=pl.BlockSpec((_FB, D, O), lambda i: (i, 0, 0)),
        out_shape=jax.ShapeDtypeStruct((F, D, O), jnp.float32),
    )(positions, obs_positions, sensor_raw, W, b2)


# speedup vs baseline: 1.5377x; 1.1078x over previous
import math

import jax
import jax.numpy as jnp
from jax.experimental import pallas as pl
from jax.experimental.pallas import tpu as pltpu

F = 2048
O = 128
D = 128
NUM_SENSORS = 100
X_MIN = -2.5
X_MAX = 2.5
PNR = 10.0
EXPECTED = 8.0

_BW = (X_MAX - X_MIN) / NUM_SENSORS
_NP = 10.0 ** (-PNR / 10.0)
_INV2BW2 = 1.0 / (2.0 * _BW * _BW)
_INV2NP2 = 1.0 / (2.0 * _NP * _NP)
_REPL = max(1.0, O / EXPECTED)
_CONST = (-math.log(_BW) - 0.5 * math.log(2.0 * math.pi) + math.log(EXPECTED)
          + math.log(X_MAX - X_MIN) - math.log(D - EXPECTED) - math.log(_REPL))

_FB = 32


def _tc_body(pos_ref, obs_ref, sen_ref, w_ref, b_ref, out_ref):
    w = w_ref[0, 0]
    bb = b_ref[0, 0]
    pos = pos_ref[...]
    obs = obs_ref[...]
    conf = jax.nn.sigmoid((sen_ref[...] - 0.5) * w + bb)
    ct = (2.0 * conf - 1.0) * _INV2NP2 + _CONST
    a_row = ct - _INV2BW2 * obs * obs
    b_row = (2.0 * _INV2BW2) * obs
    lhs = jnp.stack([a_row, b_row, jnp.ones_like(obs)], axis=1)
    rhs = jnp.stack([jnp.ones_like(pos), pos, -_INV2BW2 * pos * pos], axis=1)
    out_ref[...] = jax.lax.dot_general(
        lhs, rhs, (((1,), (1,)), ((0,), (0,))),
        preferred_element_type=jnp.float32)


def kernel(positions, obs_positions, sensor_raw, W, b):
    b2 = b.reshape(1, 1)
    grid = (F // _FB,)
    return pl.pallas_call(
        _tc_body,
        grid=grid,
        in_specs=[
            pl.BlockSpec((_FB, O), lambda i: (i, 0)),
            pl.BlockSpec((_FB, D), lambda i: (i, 0)),
            pl.BlockSpec((_FB, D), lambda i: (i, 0)),
            pl.BlockSpec(memory_space=pltpu.SMEM),
            pl.BlockSpec(memory_space=pltpu.SMEM),
        ],
        out_specs=pl.BlockSpec((_FB, D, O), lambda i: (i, 0, 0)),
        out_shape=jax.ShapeDtypeStruct((F, D, O), jnp.float32),
    )(positions, obs_positions, sensor_raw, W, b2)
